# baseline (speedup 1.0000x reference)
import jax
import jax.numpy as jnp
from jax.experimental import pallas as pl
from jax.experimental.pallas import tpu as pltpu

K_KNN = 4
LAMBDA_FUSE = 0.1
BALL_WEIGHT = 0.5
TAU_CENTER = 0.35
EPS = 1e-6

_M = 6
_LANES = 1920
_ROWS = 640
_CH_ROWS = 10
_NORM = 1.0 / (128 * 25)


def _taglayer_body(x_ref, y_ref):
    xb = x_ref[0]

    head = xb[: 4 * _CH_ROWS].reshape(4, _CH_ROWS, _LANES)
    csum = jnp.sum(head, axis=1)
    lane_m = jax.lax.broadcasted_iota(jnp.int32, (_M, _LANES), 1) % _M
    onehot6 = (lane_m == jax.lax.broadcasted_iota(
        jnp.int32, (_M, _LANES), 0)).astype(jnp.float32)
    smat = jax.lax.dot_general(
        csum, onehot6, dimension_numbers=(((1,), (1,)), ((), ())),
        preferred_element_type=jnp.float32) * _NORM
    pos = smat[:3]
    ball = smat[3:4]

    diff = pos[:, :, None] - pos[:, None, :]
    d = jnp.sqrt(jnp.sum(diff * diff, axis=0) + 1e-12)

    sneg = -d
    li = jax.lax.broadcasted_iota(jnp.int32, (_M, _M, _M), 2)
    ji = jax.lax.broadcasted_iota(jnp.int32, (_M, _M, _M), 1)
    t = sneg[:, None, :]
    v = sneg[:, :, None]
    better = (t > v) | ((t == v) & (li < ji))
    rank = jnp.sum(better.astype(jnp.int32), axis=-1)
    k_eff = max(1, min(int(K_KNN), _M))
    ui = jax.lax.broadcasted_iota(jnp.int32, (_M, _M), 0)
    mi = jax.lax.broadcasted_iota(jnp.int32, (_M, _M), 1)
    eye = (ui == mi).astype(jnp.float32)
    a_knn = (rank < k_eff).astype(jnp.float32) + eye

    tau = max(1e-6, float(TAU_CENTER))
    logits = ball * (1.0 / tau)
    z = jnp.exp(logits - jnp.max(logits, axis=1, keepdims=True))
    p = z / jnp.sum(z, axis=1, keepdims=True)
    a_ball = p.T + p + eye

    a = BALL_WEIGHT * a_ball + (1.0 - BALL_WEIGHT) * a_knn
    drow = jnp.sum(a, axis=-1, keepdims=True)
    dis = jax.lax.rsqrt(drow + EPS)
    a = dis * a * dis.T

    g = eye + LAMBDA_FUSE * a.T
    gds = []
    for dd in range(-(_M - 1), _M):
        sel = (mi - ui == dd).astype(jnp.float32)
        gds.append(jnp.sum(g * sel, axis=0, keepdims=True))
    d11 = jnp.concatenate(gds, axis=0)
    wtab = jax.lax.dot_general(
        d11, onehot6, dimension_numbers=(((1,), (0,)), ((), ())),
        preferred_element_type=jnp.float32)

    acc = xb * wtab[_M - 1: _M, :]
    for dd in range(-(_M - 1), _M):
        if dd == 0:
            continue
        acc = acc + (pltpu.roll(xb, dd % _LANES, 1)
                     * wtab[dd + _M - 1: dd + _M, :])
    y_ref[0] = acc


def kernel(x):
    N, C, T, V, M = x.shape
    x3 = x.reshape(N, _ROWS, _LANES)
    y3 = pl.pallas_call(
        _taglayer_body,
        grid=(N,),
        in_specs=[pl.BlockSpec((1, _ROWS, _LANES), lambda n: (n, 0, 0))],
        out_specs=pl.BlockSpec((1, _ROWS, _LANES), lambda n: (n, 0, 0)),
        out_shape=jax.ShapeDtypeStruct((N, _ROWS, _LANES), x.dtype),
    )(x3)
    return y3.reshape(N, C, T, V, M)


# speedup vs baseline: 2.6282x; 2.6282x over previous
import jax
import jax.numpy as jnp
from jax.experimental import pallas as pl
from jax.experimental.pallas import tpu as pltpu

K_KNN = 4
LAMBDA_FUSE = 0.1
BALL_WEIGHT = 0.5
TAU_CENTER = 0.35
EPS = 1e-6

_M = 6
_LANES = 150
_ROWS = 8192
_RB = 1024
_STAT_ROWS = 512
_NORM = 1.0 / (128 * 25)


def _compute_bfull(xs):
    csum = jnp.sum(xs.reshape(4, 128, _LANES), axis=1)
    lane6 = jax.lax.broadcasted_iota(jnp.int32, (_M, _LANES), 1) % _M
    onehot6 = (lane6 == jax.lax.broadcasted_iota(
        jnp.int32, (_M, _LANES), 0)).astype(jnp.float32)
    smat = jax.lax.dot_general(
        csum, onehot6, dimension_numbers=(((1,), (1,)), ((), ())),
        preferred_element_type=jnp.float32) * _NORM
    pos = smat[:3]
    ball = smat[3:4]

    diff = pos[:, :, None] - pos[:, None, :]
    d = jnp.sqrt(jnp.sum(diff * diff, axis=0) + 1e-12)

    sneg = -d
    li = jax.lax.broadcasted_iota(jnp.int32, (_M, _M, _M), 2)
    ji = jax.lax.broadcasted_iota(jnp.int32, (_M, _M, _M), 1)
    better = ((sneg[:, None, :] > sneg[:, :, None])
              | ((sneg[:, None, :] == sneg[:, :, None]) & (li < ji)))
    rank = jnp.sum(better.astype(jnp.int32), axis=-1)
    k_eff = max(1, min(int(K_KNN), _M))
    ui = jax.lax.broadcasted_iota(jnp.int32, (_M, _M), 0)
    mi = jax.lax.broadcasted_iota(jnp.int32, (_M, _M), 1)
    eye = (ui == mi).astype(jnp.float32)
    a_knn = (rank < k_eff).astype(jnp.float32) + eye

    tau = max(1e-6, float(TAU_CENTER))
    logits = ball * (1.0 / tau)
    z = jnp.exp(logits - jnp.max(logits, axis=1, keepdims=True))
    p = z / jnp.sum(z, axis=1, keepdims=True)
    a_ball = p.T + p + eye

    a = BALL_WEIGHT * a_ball + (1.0 - BALL_WEIGHT) * a_knn
    drow = jnp.sum(a, axis=-1, keepdims=True)
    dis = jax.lax.rsqrt(drow + EPS)
    a = dis * a * dis.T

    g = eye + LAMBDA_FUSE * a.T

    oh_t = (jax.lax.broadcasted_iota(jnp.int32, (_LANES, _M), 0) % _M
            == jax.lax.broadcasted_iota(
                jnp.int32, (_LANES, _M), 1)).astype(jnp.float32)
    tmp = jax.lax.dot_general(
        oh_t, g, dimension_numbers=(((1,), (0,)), ((), ())),
        preferred_element_type=jnp.float32)
    g_big = jax.lax.dot_general(
        tmp, onehot6, dimension_numbers=(((1,), (0,)), ((), ())),
        preferred_element_type=jnp.float32)
    ri = jax.lax.broadcasted_iota(jnp.int32, (_LANES, _LANES), 0)
    ci = jax.lax.broadcasted_iota(jnp.int32, (_LANES, _LANES), 1)
    blockmask = ((ri // _M) == (ci // _M)).astype(jnp.float32)
    return g_big * blockmask


def _taglayer_body(x_ref, y_ref, b_ref):
    r = pl.program_id(1)

    @pl.when(r == 0)
    def _():
        b_ref[...] = _compute_bfull(x_ref[0, :_STAT_ROWS])

    y_ref[0] = jax.lax.dot_general(
        x_ref[0], b_ref[...],
        dimension_numbers=(((1,), (0,)), ((), ())),
        preferred_element_type=jnp.float32)


def kernel(x):
    N, C, T, V, M = x.shape
    x3 = x.reshape(N, _ROWS, _LANES)
    y3 = pl.pallas_call(
        _taglayer_body,
        grid=(N, _ROWS // _RB),
        in_specs=[pl.BlockSpec((1, _RB, _LANES), lambda n, r: (n, r, 0))],
        out_specs=pl.BlockSpec((1, _RB, _LANES), lambda n, r: (n, r, 0)),
        out_shape=jax.ShapeDtypeStruct((N, _ROWS, _LANES), x.dtype),
        scratch_shapes=[pltpu.VMEM((_LANES, _LANES), jnp.float32)],
    )(x3)
    return y3.reshape(N, C, T, V, M)
